# baseline (speedup 1.0000x reference)
import functools

import jax
import jax.numpy as jnp
from jax import lax
from jax.experimental import pallas as pl
from jax.experimental.pallas import tpu as pltpu
from jax.experimental.pallas import tpu_sc as plsc

N_NODES = 10000
N_EDGES = 160000
N_ATOM_BASIS = 256
N_FILTERS = 256
N_GAUSSIANS = 64
CUTOFF = 5.0
LOG2 = 0.6931471805599453

HALF = N_FILTERS // 2

NT = 16
EPT = N_EDGES // NT
EB = 40
NB = EPT // EB
ROWS_PT = 640
N_PAD = ROWS_PT * NT

BE = 2000
BN = 2000


def _ssp(x):
    return jax.nn.softplus(x) - LOG2


def _rf_body(r_ref, waf_ref, lo_ref, hi_ref):
    rf = jnp.dot(r_ref[...], waf_ref[...], preferred_element_type=jnp.float32)
    lo_ref[...] = rf[:, :HALF]
    hi_ref[...] = rf[:, HALF:]


def _rf_call(r, w_af):
    grid = (N_NODES // BN,)
    return pl.pallas_call(
        _rf_body,
        grid=grid,
        in_specs=[
            pl.BlockSpec((BN, N_ATOM_BASIS), lambda i: (i, 0)),
            pl.BlockSpec((N_ATOM_BASIS, N_FILTERS), lambda i: (0, 0)),
        ],
        out_specs=[
            pl.BlockSpec((BN, HALF), lambda i: (i, 0)),
            pl.BlockSpec((BN, HALF), lambda i: (i, 0)),
        ],
        out_shape=[
            jax.ShapeDtypeStruct((N_NODES, HALF), jnp.float32),
            jax.ShapeDtypeStruct((N_NODES, HALF), jnp.float32),
        ],
    )(r, w_af)


def _edge_mlp_body(e_ref, w1_ref, b1_ref, w2_ref, b2_ref, lo_ref, hi_ref):
    e = e_ref[...]
    step = CUTOFF / (N_GAUSSIANS - 1)
    offs = lax.broadcasted_iota(jnp.int32, (1, N_GAUSSIANS), 1).astype(jnp.float32) * step
    diff = e - offs
    coeff = -0.5 / (step * step)
    eg = jnp.exp(coeff * diff * diff)
    h = jnp.dot(eg, w1_ref[...], preferred_element_type=jnp.float32) + b1_ref[...]
    h = _ssp(h)
    w = jnp.dot(h, w2_ref[...], preferred_element_type=jnp.float32) + b2_ref[...]
    lo_ref[...] = w[:, :HALF]
    hi_ref[...] = w[:, HALF:]


def _edge_mlp_call(e, w_df1, b_df1, w_df2, b_df2):
    grid = (N_EDGES // BE,)
    return pl.pallas_call(
        _edge_mlp_body,
        grid=grid,
        in_specs=[
            pl.BlockSpec((BE, 1), lambda i: (i, 0)),
            pl.BlockSpec((N_GAUSSIANS, N_GAUSSIANS), lambda i: (0, 0)),
            pl.BlockSpec((1, N_GAUSSIANS), lambda i: (0, 0)),
            pl.BlockSpec((N_GAUSSIANS, N_FILTERS), lambda i: (0, 0)),
            pl.BlockSpec((1, N_FILTERS), lambda i: (0, 0)),
        ],
        out_specs=[
            pl.BlockSpec((BE, HALF), lambda i: (i, 0)),
            pl.BlockSpec((BE, HALF), lambda i: (i, 0)),
        ],
        out_shape=[
            jax.ShapeDtypeStruct((N_EDGES, HALF), jnp.float32),
            jax.ShapeDtypeStruct((N_EDGES, HALF), jnp.float32),
        ],
    )(e, w_df1, b_df1, w_df2, b_df2)


def _sc_body(rfl, rfh, wl, wh, sd, yl, yh,
             idx_v, rfs_v, rfd_v, w_v, acc_sh,
             gi, gs, gd, gw, ssm, sdm):
    c = lax.axis_index("c")
    s = lax.axis_index("s")

    def zero_row(i, carry):
        for j in range(HALF // 16):
            rfs_v[0, i, pl.ds(j * 16, 16)] = jnp.zeros((16,), jnp.float32)
        return carry

    lax.fori_loop(0, EB, zero_row, 0)

    def zero_chunk(k, carry):
        pltpu.sync_copy(rfs_v.at[0], acc_sh.at[pl.ds(s * ROWS_PT + k * EB, EB)])
        return carry

    lax.fori_loop(0, ROWS_PT // EB, zero_chunk, 0)
    plsc.subcore_barrier()

    def work(rf_hbm, w_hbm, y_hbm):
        def issue_idx(b, slot):
            pltpu.async_copy(sd.at[s, b], idx_v.at[slot], gi.at[slot])

        def issue_gathers(b, slot, par):
            pltpu.make_async_copy(sd.at[s, b], idx_v.at[slot], gi.at[slot]).wait()
            pltpu.async_copy(rf_hbm.at[idx_v.at[slot, 0]], rfs_v.at[par], gs.at[par])
            pltpu.async_copy(rf_hbm.at[idx_v.at[slot, 1]], rfd_v.at[par], gd.at[par])
            pltpu.async_copy(w_hbm.at[pl.ds(s * EPT + b * EB, EB)],
                             w_v.at[par], gw.at[par])

        def wait_scatters(par):
            pltpu.make_async_copy(rfs_v.at[par], acc_sh.at[idx_v.at[0, 1]],
                                  ssm.at[par]).wait()
            pltpu.make_async_copy(rfd_v.at[par], acc_sh.at[idx_v.at[0, 0]],
                                  sdm.at[par]).wait()

        issue_idx(0, 0)
        issue_idx(1, 1)
        issue_gathers(0, 0, 0)

        def blk_body(b, carry):
            p = b % 2
            q = 1 - p
            i1 = (b + 1) % 3
            i2 = (b + 2) % 3

            @pl.when(b >= 1)
            def _():
                wait_scatters(q)

            @pl.when(b + 2 < NB)
            def _():
                issue_idx(b + 2, i2)

            @pl.when(b + 1 < NB)
            def _():
                issue_gathers(b + 1, i1, q)

            pltpu.make_async_copy(rf_hbm.at[idx_v.at[0, 0]], rfs_v.at[p],
                                  gs.at[p]).wait()
            pltpu.make_async_copy(rf_hbm.at[idx_v.at[0, 1]], rfd_v.at[p],
                                  gd.at[p]).wait()
            pltpu.make_async_copy(w_hbm.at[pl.ds(0, EB)], w_v.at[p],
                                  gw.at[p]).wait()

            def mul_body(i, carry2):
                for j in range(HALF // 16):
                    sl = pl.ds(j * 16, 16)
                    wv = w_v[p, i, sl]
                    rfs_v[p, i, sl] = rfs_v[p, i, sl] * wv
                    rfd_v[p, i, sl] = rfd_v[p, i, sl] * wv
                return carry2

            lax.fori_loop(0, EB, mul_body, 0)

            ip = b % 3
            pltpu.async_copy(rfs_v.at[p], acc_sh.at[idx_v.at[ip, 1]],
                             ssm.at[p], add=True)
            pltpu.async_copy(rfd_v.at[p], acc_sh.at[idx_v.at[ip, 0]],
                             sdm.at[p], add=True)
            return carry

        lax.fori_loop(0, NB, blk_body, 0)
        wait_scatters((NB - 1) % 2)
        plsc.subcore_barrier()

        def out_chunk(k, carry):
            base = s * ROWS_PT + k * EB
            pltpu.sync_copy(acc_sh.at[pl.ds(base, EB)], rfs_v.at[0])
            pltpu.sync_copy(rfs_v.at[0], y_hbm.at[pl.ds(base, EB)])
            return carry

        lax.fori_loop(0, ROWS_PT // EB, out_chunk, 0)

    @pl.when(c == 0)
    def _():
        work(rfl, wl, yl)

    @pl.when(c == 1)
    def _():
        work(rfh, wh, yh)


def _sc_call(rf_lo, rf_hi, w_lo, w_hi, sd):
    mesh = plsc.VectorSubcoreMesh(core_axis_name="c", subcore_axis_name="s",
                                  num_cores=2, num_subcores=NT)
    f = functools.partial(
        pl.kernel,
        out_type=[
            jax.ShapeDtypeStruct((N_PAD, HALF), jnp.float32),
            jax.ShapeDtypeStruct((N_PAD, HALF), jnp.float32),
        ],
        mesh=mesh,
        scratch_types=[
            pltpu.VMEM((3, 2, EB), jnp.int32),
            pltpu.VMEM((2, EB, HALF), jnp.float32),
            pltpu.VMEM((2, EB, HALF), jnp.float32),
            pltpu.VMEM((2, EB, HALF), jnp.float32),
            pltpu.VMEM_SHARED((N_PAD, HALF), jnp.float32),
            pltpu.SemaphoreType.DMA((3,)),
            pltpu.SemaphoreType.DMA((2,)),
            pltpu.SemaphoreType.DMA((2,)),
            pltpu.SemaphoreType.DMA((2,)),
            pltpu.SemaphoreType.DMA((2,)),
            pltpu.SemaphoreType.DMA((2,)),
        ],
    )(_sc_body)
    return f(rf_lo, rf_hi, w_lo, w_hi, sd)


def _out_mlp_body(ylo_ref, yhi_ref, w1_ref, b1_ref, w2_ref, b2_ref, o_ref):
    y = jnp.concatenate([ylo_ref[...], yhi_ref[...]], axis=1)
    h = jnp.dot(y, w1_ref[...], preferred_element_type=jnp.float32) + b1_ref[...]
    h = _ssp(h)
    o_ref[...] = jnp.dot(h, w2_ref[...], preferred_element_type=jnp.float32) + b2_ref[...]


def _out_mlp_call(y_lo, y_hi, w_d1, b_d1, w_d2, b_d2):
    grid = (N_NODES // BN,)
    return pl.pallas_call(
        _out_mlp_body,
        grid=grid,
        in_specs=[
            pl.BlockSpec((BN, HALF), lambda i: (i, 0)),
            pl.BlockSpec((BN, HALF), lambda i: (i, 0)),
            pl.BlockSpec((N_FILTERS, N_ATOM_BASIS), lambda i: (0, 0)),
            pl.BlockSpec((1, N_ATOM_BASIS), lambda i: (0, 0)),
            pl.BlockSpec((N_ATOM_BASIS, N_ATOM_BASIS), lambda i: (0, 0)),
            pl.BlockSpec((1, N_ATOM_BASIS), lambda i: (0, 0)),
        ],
        out_specs=pl.BlockSpec((BN, N_ATOM_BASIS), lambda i: (i, 0)),
        out_shape=jax.ShapeDtypeStruct((N_NODES, N_ATOM_BASIS), jnp.float32),
    )(y_lo, y_hi, w_d1, b_d1, w_d2, b_d2)


def kernel(r, e, a, W_df1, b_df1, W_df2, b_df2, W_af, W_d1, b_d1, W_d2, b_d2):
    src = a[:, 0].reshape(NT, NB, EB)
    dst = a[:, 1].reshape(NT, NB, EB)
    sd = jnp.stack([src, dst], axis=2)
    rf_lo, rf_hi = _rf_call(r, W_af)
    w_lo, w_hi = _edge_mlp_call(e, W_df1, b_df1.reshape(1, -1),
                                W_df2, b_df2.reshape(1, -1))
    y_lo, y_hi = _sc_call(rf_lo, rf_hi, w_lo, w_hi, sd)
    y_lo = y_lo[:N_NODES]
    y_hi = y_hi[:N_NODES]
    return _out_mlp_call(y_lo, y_hi, W_d1, b_d1.reshape(1, -1),
                         W_d2, b_d2.reshape(1, -1))


# speedup vs baseline: 2.2477x; 2.2477x over previous
import functools

import jax
import jax.numpy as jnp
from jax import lax
from jax.experimental import pallas as pl
from jax.experimental.pallas import tpu as pltpu
from jax.experimental.pallas import tpu_sc as plsc

N_NODES = 10000
N_EDGES = 160000
N_ATOM_BASIS = 256
N_FILTERS = 256
N_GAUSSIANS = 64
CUTOFF = 5.0
LOG2 = 0.6931471805599453

HALF = N_FILTERS // 2

NT = 16
EPT = N_EDGES // NT
EB = 40
NB = EPT // EB
ROWS_PT = 640
N_PAD = ROWS_PT * NT

BE = 2000
BN = 2000


def _ssp(x):
    return jax.nn.softplus(x) - LOG2


def _rf_body(r_ref, waf_ref, lo_ref, hi_ref):
    rf = jnp.dot(r_ref[...], waf_ref[...], preferred_element_type=jnp.float32)
    lo_ref[...] = rf[:, :HALF]
    hi_ref[...] = rf[:, HALF:]


def _rf_call(r, w_af):
    grid = (N_NODES // BN,)
    return pl.pallas_call(
        _rf_body,
        grid=grid,
        in_specs=[
            pl.BlockSpec((BN, N_ATOM_BASIS), lambda i: (i, 0)),
            pl.BlockSpec((N_ATOM_BASIS, N_FILTERS), lambda i: (0, 0)),
        ],
        out_specs=[
            pl.BlockSpec((BN, HALF), lambda i: (i, 0)),
            pl.BlockSpec((BN, HALF), lambda i: (i, 0)),
        ],
        out_shape=[
            jax.ShapeDtypeStruct((N_NODES, HALF), jnp.float32),
            jax.ShapeDtypeStruct((N_NODES, HALF), jnp.float32),
        ],
    )(r, w_af)


def _edge_mlp_body(e_ref, w1_ref, b1_ref, w2_ref, b2_ref, lo_ref, hi_ref):
    e = e_ref[...]
    step = CUTOFF / (N_GAUSSIANS - 1)
    offs = lax.broadcasted_iota(jnp.int32, (1, N_GAUSSIANS), 1).astype(jnp.float32) * step
    diff = e - offs
    coeff = -0.5 / (step * step)
    eg = jnp.exp(coeff * diff * diff)
    h = jnp.dot(eg, w1_ref[...], preferred_element_type=jnp.float32) + b1_ref[...]
    h = _ssp(h)
    w = jnp.dot(h, w2_ref[...], preferred_element_type=jnp.float32) + b2_ref[...]
    lo_ref[...] = w[:, :HALF]
    hi_ref[...] = w[:, HALF:]


def _edge_mlp_call(e, w_df1, b_df1, w_df2, b_df2):
    grid = (N_EDGES // BE,)
    return pl.pallas_call(
        _edge_mlp_body,
        grid=grid,
        in_specs=[
            pl.BlockSpec((BE, 1), lambda i: (i, 0)),
            pl.BlockSpec((N_GAUSSIANS, N_GAUSSIANS), lambda i: (0, 0)),
            pl.BlockSpec((1, N_GAUSSIANS), lambda i: (0, 0)),
            pl.BlockSpec((N_GAUSSIANS, N_FILTERS), lambda i: (0, 0)),
            pl.BlockSpec((1, N_FILTERS), lambda i: (0, 0)),
        ],
        out_specs=[
            pl.BlockSpec((BE, HALF), lambda i: (i, 0)),
            pl.BlockSpec((BE, HALF), lambda i: (i, 0)),
        ],
        out_shape=[
            jax.ShapeDtypeStruct((N_EDGES, HALF), jnp.float32),
            jax.ShapeDtypeStruct((N_EDGES, HALF), jnp.float32),
        ],
    )(e, w_df1, b_df1, w_df2, b_df2)


EB2 = 2 * EB
UNROLL = 10
NGRP = NB // UNROLL
CHUNK = EB2


def _sc_body(rfl, rfh, wl, wh, sd, yl, yh,
             ring, rfsd_a, rfsd_b, w_a, w_b, acc_sh,
             gi, gg, gw, ssc):
    c = lax.axis_index("c")
    s = lax.axis_index("s")

    def zero_row(i, carry):
        for j in range(HALF // 16):
            rfsd_a[i, pl.ds(j * 16, 16)] = jnp.zeros((16,), jnp.float32)
        return carry

    lax.fori_loop(0, EB2, zero_row, 0)

    def zero_chunk(k, carry):
        pltpu.sync_copy(rfsd_a, acc_sh.at[pl.ds(s * ROWS_PT + k * CHUNK, CHUNK)])
        return carry

    lax.fori_loop(0, ROWS_PT // CHUNK, zero_chunk, 0)
    plsc.subcore_barrier()

    bufs = ((rfsd_a, w_a), (rfsd_b, w_b))

    def work(rf_hbm, w_hbm, y_hbm):
        def idx_issue(b, slot):
            pltpu.async_copy(sd.at[s, b], ring.at[slot], gi.at[slot])

        def idx_wait(b, slot):
            pltpu.make_async_copy(sd.at[s, b], ring.at[slot],
                                  gi.at[slot]).wait()

        def gather_issue(b, slot, rfsd, wv, par):
            pltpu.async_copy(rf_hbm.at[ring.at[slot, 0]], rfsd, gg.at[par])
            pltpu.async_copy(w_hbm.at[pl.ds(s * EPT + b * EB, EB)], wv,
                             gw.at[par])

        def gather_wait(rfsd, wv, par):
            pltpu.make_async_copy(rf_hbm.at[ring.at[0, 0]], rfsd,
                                  gg.at[par]).wait()
            pltpu.make_async_copy(w_hbm.at[pl.ds(0, EB)], wv,
                                  gw.at[par]).wait()

        def scat_issue(slot, rfsd, par):
            pltpu.async_copy(rfsd, acc_sh.at[ring.at[slot, 1]], ssc.at[par],
                             add=True)

        def scat_wait(rfsd, par):
            pltpu.make_async_copy(rfsd, acc_sh.at[ring.at[0, 1]],
                                  ssc.at[par]).wait()

        def mul(rfsd, wv):
            def body(i, carry):
                for j in range(HALF // 16):
                    sl = pl.ds(j * 16, 16)
                    w16 = wv[i, sl]
                    rfsd[i, sl] = rfsd[i, sl] * w16
                    rfsd[EB + i, sl] = rfsd[EB + i, sl] * w16
                return carry

            lax.fori_loop(0, EB, body, 0)

        idx_issue(0, 0)
        idx_issue(1, 1)
        idx_wait(0, 0)
        gather_issue(0, 0, rfsd_a, w_a, 0)

        def grp_body(m, carry):
            for t in range(UNROLL):
                b = m * UNROLL + t
                p = t % 2
                q = (t + 1) % 2
                slot = t % 5
                slot1 = (t + 1) % 5
                slot2 = (t + 2) % 5
                rfsd_p, w_p = bufs[p]
                rfsd_q, w_q = bufs[q]

                if t == 0:
                    @pl.when(m > 0)
                    def _(q=q):
                        scat_wait(bufs[q][0], q)
                else:
                    scat_wait(rfsd_q, q)

                if t < UNROLL - 1:
                    idx_wait(b + 1, slot1)
                    gather_issue(b + 1, slot1, rfsd_q, w_q, q)
                else:
                    @pl.when(m < NGRP - 1)
                    def _(b=b, slot1=slot1, rfsd_q=rfsd_q, w_q=w_q, q=q):
                        idx_wait(b + 1, slot1)
                        gather_issue(b + 1, slot1, rfsd_q, w_q, q)

                if t < UNROLL - 2:
                    idx_issue(b + 2, slot2)
                else:
                    @pl.when(m < NGRP - 1)
                    def _(b=b, slot2=slot2):
                        idx_issue(b + 2, slot2)

                gather_wait(rfsd_p, w_p, p)
                mul(rfsd_p, w_p)
                scat_issue(slot, rfsd_p, p)
            return carry

        lax.fori_loop(0, NGRP, grp_body, 0)
        scat_wait(bufs[(NB - 1) % 2][0], (NB - 1) % 2)
        plsc.subcore_barrier()

        def out_chunk(k, carry):
            base = s * ROWS_PT + k * CHUNK
            pltpu.sync_copy(acc_sh.at[pl.ds(base, CHUNK)], rfsd_a)
            pltpu.sync_copy(rfsd_a, y_hbm.at[pl.ds(base, CHUNK)])
            return carry

        lax.fori_loop(0, ROWS_PT // CHUNK, out_chunk, 0)

    @pl.when(c == 0)
    def _():
        work(rfl, wl, yl)

    @pl.when(c == 1)
    def _():
        work(rfh, wh, yh)


def _sc_call(rf_lo, rf_hi, w_lo, w_hi, sd):
    mesh = plsc.VectorSubcoreMesh(core_axis_name="c", subcore_axis_name="s",
                                  num_cores=2, num_subcores=NT)
    f = functools.partial(
        pl.kernel,
        out_type=[
            jax.ShapeDtypeStruct((N_PAD, HALF), jnp.float32),
            jax.ShapeDtypeStruct((N_PAD, HALF), jnp.float32),
        ],
        mesh=mesh,
        scratch_types=[
            pltpu.VMEM((5, 2, EB2), jnp.int32),
            pltpu.VMEM((EB2, HALF), jnp.float32),
            pltpu.VMEM((EB2, HALF), jnp.float32),
            pltpu.VMEM((EB, HALF), jnp.float32),
            pltpu.VMEM((EB, HALF), jnp.float32),
            pltpu.VMEM_SHARED((N_PAD, HALF), jnp.float32),
            pltpu.SemaphoreType.DMA((5,)),
            pltpu.SemaphoreType.DMA((2,)),
            pltpu.SemaphoreType.DMA((2,)),
            pltpu.SemaphoreType.DMA((2,)),
        ],
    )(_sc_body)
    return f(rf_lo, rf_hi, w_lo, w_hi, sd)


def _out_mlp_body(ylo_ref, yhi_ref, w1_ref, b1_ref, w2_ref, b2_ref, o_ref):
    y = jnp.concatenate([ylo_ref[...], yhi_ref[...]], axis=1)
    h = jnp.dot(y, w1_ref[...], preferred_element_type=jnp.float32) + b1_ref[...]
    h = _ssp(h)
    o_ref[...] = jnp.dot(h, w2_ref[...], preferred_element_type=jnp.float32) + b2_ref[...]


def _out_mlp_call(y_lo, y_hi, w_d1, b_d1, w_d2, b_d2):
    grid = (N_NODES // BN,)
    return pl.pallas_call(
        _out_mlp_body,
        grid=grid,
        in_specs=[
            pl.BlockSpec((BN, HALF), lambda i: (i, 0)),
            pl.BlockSpec((BN, HALF), lambda i: (i, 0)),
            pl.BlockSpec((N_FILTERS, N_ATOM_BASIS), lambda i: (0, 0)),
            pl.BlockSpec((1, N_ATOM_BASIS), lambda i: (0, 0)),
            pl.BlockSpec((N_ATOM_BASIS, N_ATOM_BASIS), lambda i: (0, 0)),
            pl.BlockSpec((1, N_ATOM_BASIS), lambda i: (0, 0)),
        ],
        out_specs=pl.BlockSpec((BN, N_ATOM_BASIS), lambda i: (i, 0)),
        out_shape=jax.ShapeDtypeStruct((N_NODES, N_ATOM_BASIS), jnp.float32),
    )(y_lo, y_hi, w_d1, b_d1, w_d2, b_d2)


def kernel(r, e, a, W_df1, b_df1, W_df2, b_df2, W_af, W_d1, b_d1, W_d2, b_d2):
    src = a[:, 0].reshape(NT, NB, EB)
    dst = a[:, 1].reshape(NT, NB, EB)
    fwd = jnp.concatenate([src, dst], axis=-1)
    bwd = jnp.concatenate([dst, src], axis=-1)
    sd = jnp.stack([fwd, bwd], axis=2)
    rf_lo, rf_hi = _rf_call(r, W_af)
    w_lo, w_hi = _edge_mlp_call(e, W_df1, b_df1.reshape(1, -1),
                                W_df2, b_df2.reshape(1, -1))
    y_lo, y_hi = _sc_call(rf_lo, rf_hi, w_lo, w_hi, sd)
    y_lo = y_lo[:N_NODES]
    y_hi = y_hi[:N_NODES]
    return _out_mlp_call(y_lo, y_hi, W_d1, b_d1.reshape(1, -1),
                         W_d2, b_d2.reshape(1, -1))


# speedup vs baseline: 2.2946x; 1.0208x over previous
import functools

import jax
import jax.numpy as jnp
from jax import lax
from jax.experimental import pallas as pl
from jax.experimental.pallas import tpu as pltpu
from jax.experimental.pallas import tpu_sc as plsc

N_NODES = 10000
N_EDGES = 160000
N_ATOM_BASIS = 256
N_FILTERS = 256
N_GAUSSIANS = 64
CUTOFF = 5.0
LOG2 = 0.6931471805599453

HALF = N_FILTERS // 2

NT = 16
EPT = N_EDGES // NT
EB = 40
NB = EPT // EB
ROWS_PT = 640
N_PAD = ROWS_PT * NT

BE = 2000
BN = 2000


def _ssp(x):
    return jax.nn.softplus(x) - LOG2


NRB = N_NODES // BE


def _edge_mlp_body(e_ref, w1_ref, b1_ref, w2_ref, b2_ref, r_ref, waf_ref,
                   lo_ref, hi_ref, rlo_ref, rhi_ref):
    e = e_ref[...]
    step = CUTOFF / (N_GAUSSIANS - 1)
    offs = lax.broadcasted_iota(jnp.int32, (1, N_GAUSSIANS), 1).astype(jnp.float32) * step
    diff = e - offs
    coeff = -0.5 / (step * step)
    eg = jnp.exp(coeff * diff * diff)
    h = jnp.dot(eg, w1_ref[...], preferred_element_type=jnp.float32) + b1_ref[...]
    h = _ssp(h)
    w = jnp.dot(h, w2_ref[...], preferred_element_type=jnp.float32) + b2_ref[...]
    lo_ref[...] = w[:, :HALF]
    hi_ref[...] = w[:, HALF:]

    @pl.when(pl.program_id(0) < NRB)
    def _():
        rf = jnp.dot(r_ref[...], waf_ref[...],
                     preferred_element_type=jnp.float32)
        rlo_ref[...] = rf[:, :HALF]
        rhi_ref[...] = rf[:, HALF:]


def _edge_mlp_call(e, w_df1, b_df1, w_df2, b_df2, r, w_af):
    grid = (N_EDGES // BE,)
    node_map = lambda i: (jnp.minimum(i, NRB - 1), 0)
    return pl.pallas_call(
        _edge_mlp_body,
        grid=grid,
        in_specs=[
            pl.BlockSpec((BE, 1), lambda i: (i, 0)),
            pl.BlockSpec((N_GAUSSIANS, N_GAUSSIANS), lambda i: (0, 0)),
            pl.BlockSpec((1, N_GAUSSIANS), lambda i: (0, 0)),
            pl.BlockSpec((N_GAUSSIANS, N_FILTERS), lambda i: (0, 0)),
            pl.BlockSpec((1, N_FILTERS), lambda i: (0, 0)),
            pl.BlockSpec((BE, N_ATOM_BASIS), node_map),
            pl.BlockSpec((N_ATOM_BASIS, N_FILTERS), lambda i: (0, 0)),
        ],
        out_specs=[
            pl.BlockSpec((BE, HALF), lambda i: (i, 0)),
            pl.BlockSpec((BE, HALF), lambda i: (i, 0)),
            pl.BlockSpec((BE, HALF), node_map),
            pl.BlockSpec((BE, HALF), node_map),
        ],
        out_shape=[
            jax.ShapeDtypeStruct((N_EDGES, HALF), jnp.float32),
            jax.ShapeDtypeStruct((N_EDGES, HALF), jnp.float32),
            jax.ShapeDtypeStruct((N_NODES, HALF), jnp.float32),
            jax.ShapeDtypeStruct((N_NODES, HALF), jnp.float32),
        ],
    )(e, w_df1, b_df1, w_df2, b_df2, r, w_af)


EB2 = 2 * EB
UNROLL = 10
NGRP = NB // UNROLL
CHUNK = EB2


def _sc_body(rfl, rfh, wl, wh, sd, yl, yh,
             ring, rfsd_a, rfsd_b, w_a, w_b, acc_sh,
             gi, gg, gw, ssc):
    c = lax.axis_index("c")
    s = lax.axis_index("s")

    def zero_row(i, carry):
        for j in range(HALF // 16):
            rfsd_a[i, pl.ds(j * 16, 16)] = jnp.zeros((16,), jnp.float32)
        return carry

    lax.fori_loop(0, EB2, zero_row, 0)

    def zero_chunk(k, carry):
        pltpu.sync_copy(rfsd_a, acc_sh.at[pl.ds(s * ROWS_PT + k * CHUNK, CHUNK)])
        return carry

    lax.fori_loop(0, ROWS_PT // CHUNK, zero_chunk, 0)
    plsc.subcore_barrier()

    bufs = ((rfsd_a, w_a), (rfsd_b, w_b))

    def work(rf_hbm, w_hbm, y_hbm):
        def idx_issue(b, slot):
            pltpu.async_copy(sd.at[s, b], ring.at[slot], gi.at[slot])

        def idx_wait(b, slot):
            pltpu.make_async_copy(sd.at[s, b], ring.at[slot],
                                  gi.at[slot]).wait()

        def gather_issue(b, slot, rfsd, wv, par):
            pltpu.async_copy(rf_hbm.at[ring.at[slot, 0]], rfsd, gg.at[par])
            pltpu.async_copy(w_hbm.at[pl.ds(s * EPT + b * EB, EB)], wv,
                             gw.at[par])

        def gather_wait(rfsd, wv, par):
            pltpu.make_async_copy(rf_hbm.at[ring.at[0, 0]], rfsd,
                                  gg.at[par]).wait()
            pltpu.make_async_copy(w_hbm.at[pl.ds(0, EB)], wv,
                                  gw.at[par]).wait()

        def scat_issue(slot, rfsd, par):
            pltpu.async_copy(rfsd, acc_sh.at[ring.at[slot, 1]], ssc.at[par],
                             add=True)

        def scat_wait(rfsd, par):
            pltpu.make_async_copy(rfsd, acc_sh.at[ring.at[0, 1]],
                                  ssc.at[par]).wait()

        def mul(rfsd, wv):
            def body(k, carry):
                i = k * 2
                for d in range(2):
                    for j in range(HALF // 16):
                        sl = pl.ds(j * 16, 16)
                        w16 = wv[i + d, sl]
                        rfsd[i + d, sl] = rfsd[i + d, sl] * w16
                        rfsd[EB + i + d, sl] = rfsd[EB + i + d, sl] * w16
                return carry

            lax.fori_loop(0, EB // 2, body, 0)

        idx_issue(0, 0)
        idx_issue(1, 1)
        idx_wait(0, 0)
        gather_issue(0, 0, rfsd_a, w_a, 0)

        def grp_body(m, carry):
            for t in range(UNROLL):
                b = m * UNROLL + t
                p = t % 2
                q = (t + 1) % 2
                slot = t % 5
                slot1 = (t + 1) % 5
                slot2 = (t + 2) % 5
                rfsd_p, w_p = bufs[p]
                rfsd_q, w_q = bufs[q]

                if t == 0:
                    @pl.when(m > 0)
                    def _(q=q):
                        scat_wait(bufs[q][0], q)
                else:
                    scat_wait(rfsd_q, q)

                if t < UNROLL - 1:
                    idx_wait(b + 1, slot1)
                    gather_issue(b + 1, slot1, rfsd_q, w_q, q)
                else:
                    @pl.when(m < NGRP - 1)
                    def _(b=b, slot1=slot1, rfsd_q=rfsd_q, w_q=w_q, q=q):
                        idx_wait(b + 1, slot1)
                        gather_issue(b + 1, slot1, rfsd_q, w_q, q)

                if t < UNROLL - 2:
                    idx_issue(b + 2, slot2)
                else:
                    @pl.when(m < NGRP - 1)
                    def _(b=b, slot2=slot2):
                        idx_issue(b + 2, slot2)

                gather_wait(rfsd_p, w_p, p)
                mul(rfsd_p, w_p)
                scat_issue(slot, rfsd_p, p)
            return carry

        lax.fori_loop(0, NGRP, grp_body, 0)
        scat_wait(bufs[(NB - 1) % 2][0], (NB - 1) % 2)
        plsc.subcore_barrier()

        def out_chunk(k, carry):
            base = s * ROWS_PT + k * CHUNK
            pltpu.sync_copy(acc_sh.at[pl.ds(base, CHUNK)], rfsd_a)
            pltpu.sync_copy(rfsd_a, y_hbm.at[pl.ds(base, CHUNK)])
            return carry

        lax.fori_loop(0, ROWS_PT // CHUNK, out_chunk, 0)

    @pl.when(c == 0)
    def _():
        work(rfl, wl, yl)

    @pl.when(c == 1)
    def _():
        work(rfh, wh, yh)


def _sc_call(rf_lo, rf_hi, w_lo, w_hi, sd):
    mesh = plsc.VectorSubcoreMesh(core_axis_name="c", subcore_axis_name="s",
                                  num_cores=2, num_subcores=NT)
    f = functools.partial(
        pl.kernel,
        out_type=[
            jax.ShapeDtypeStruct((N_PAD, HALF), jnp.float32),
            jax.ShapeDtypeStruct((N_PAD, HALF), jnp.float32),
        ],
        mesh=mesh,
        scratch_types=[
            pltpu.VMEM((5, 2, EB2), jnp.int32),
            pltpu.VMEM((EB2, HALF), jnp.float32),
            pltpu.VMEM((EB2, HALF), jnp.float32),
            pltpu.VMEM((EB, HALF), jnp.float32),
            pltpu.VMEM((EB, HALF), jnp.float32),
            pltpu.VMEM_SHARED((N_PAD, HALF), jnp.float32),
            pltpu.SemaphoreType.DMA((5,)),
            pltpu.SemaphoreType.DMA((2,)),
            pltpu.SemaphoreType.DMA((2,)),
            pltpu.SemaphoreType.DMA((2,)),
        ],
    )(_sc_body)
    return f(rf_lo, rf_hi, w_lo, w_hi, sd)


def _out_mlp_body(ylo_ref, yhi_ref, w1_ref, b1_ref, w2_ref, b2_ref, o_ref):
    y = jnp.concatenate([ylo_ref[...], yhi_ref[...]], axis=1)
    h = jnp.dot(y, w1_ref[...], preferred_element_type=jnp.float32) + b1_ref[...]
    h = _ssp(h)
    o_ref[...] = jnp.dot(h, w2_ref[...], preferred_element_type=jnp.float32) + b2_ref[...]


def _out_mlp_call(y_lo, y_hi, w_d1, b_d1, w_d2, b_d2):
    grid = (N_NODES // BN,)
    return pl.pallas_call(
        _out_mlp_body,
        grid=grid,
        in_specs=[
            pl.BlockSpec((BN, HALF), lambda i: (i, 0)),
            pl.BlockSpec((BN, HALF), lambda i: (i, 0)),
            pl.BlockSpec((N_FILTERS, N_ATOM_BASIS), lambda i: (0, 0)),
            pl.BlockSpec((1, N_ATOM_BASIS), lambda i: (0, 0)),
            pl.BlockSpec((N_ATOM_BASIS, N_ATOM_BASIS), lambda i: (0, 0)),
            pl.BlockSpec((1, N_ATOM_BASIS), lambda i: (0, 0)),
        ],
        out_specs=pl.BlockSpec((BN, N_ATOM_BASIS), lambda i: (i, 0)),
        out_shape=jax.ShapeDtypeStruct((N_NODES, N_ATOM_BASIS), jnp.float32),
    )(y_lo, y_hi, w_d1, b_d1, w_d2, b_d2)


def kernel(r, e, a, W_df1, b_df1, W_df2, b_df2, W_af, W_d1, b_d1, W_d2, b_d2):
    src = a[:, 0].reshape(NT, NB, EB)
    dst = a[:, 1].reshape(NT, NB, EB)
    fwd = jnp.concatenate([src, dst], axis=-1)
    bwd = jnp.concatenate([dst, src], axis=-1)
    sd = jnp.stack([fwd, bwd], axis=2)
    w_lo, w_hi, rf_lo, rf_hi = _edge_mlp_call(e, W_df1, b_df1.reshape(1, -1),
                                              W_df2, b_df2.reshape(1, -1),
                                              r, W_af)
    y_lo, y_hi = _sc_call(rf_lo, rf_hi, w_lo, w_hi, sd)
    return _out_mlp_call(y_lo, y_hi, W_d1, b_d1.reshape(1, -1),
                         W_d2, b_d2.reshape(1, -1))
